# baseline (speedup 1.0000x reference)
import functools

import jax
import jax.numpy as jnp
from jax import lax
from jax.experimental import pallas as pl
from jax.experimental.pallas import tpu as pltpu
from jax.experimental.pallas import tpu_sc as plsc

ROWS = 8192
COLS = 2048
NC = 2
NS = 16
L = 16
CW = 128
NG = COLS // CW
NV = CW // L
HALF = ROWS // 2
QTR = ROWS // 4
R = 128
NBUF = 5
U = 4

_mesh = plsc.VectorSubcoreMesh(core_axis_name="c", subcore_axis_name="s")


@functools.partial(
    pl.kernel,
    out_type=jax.ShapeDtypeStruct((ROWS, COLS), jnp.float32),
    mesh=_mesh,
    scratch_types=[
        [pltpu.VMEM((R, CW), jnp.float32) for _ in range(NBUF)],
        pltpu.VMEM((CW,), jnp.float32),
        pltpu.VMEM((CW,), jnp.float32),
        pltpu.VMEM_SHARED((NS, CW), jnp.float32),
        pltpu.SemaphoreType.DMA,
        pltpu.SemaphoreType.DMA,
    ],
)
def _cumsum_sc(x_hbm, out_hbm, bufs, psum_v, ppart_v, shared, in_sem,
               out_sem):
    c = lax.axis_index("c")
    s = lax.axis_index("s")
    g = c * (NS // 2) + lax.rem(s, NS // 2)
    h = lax.div(s, NS // 2)
    c0 = g * CW

    def stream(row0, nchunk, carries, store):
        def in_copy(ci):
            return pltpu.async_copy(
                x_hbm.at[pl.ds(row0 + ci * R, R), pl.ds(c0, CW)],
                bufs[ci % NBUF], in_sem)

        def out_copy(ci):
            return pltpu.async_copy(
                bufs[ci % NBUF],
                out_hbm.at[pl.ds(row0 + ci * R, R), pl.ds(c0, CW)], out_sem)

        prefetch = (NBUF - 2) if store else (NBUF - 1)
        h_in, h_out = {}, {}
        for ci in range(min(prefetch, nchunk)):
            h_in[ci] = in_copy(ci)
        for ci in range(nchunk):
            pi = ci + prefetch
            if pi < nchunk:
                prev = pi - NBUF
                if store and prev >= 0:
                    h_out.pop(prev).wait()
                h_in[pi] = in_copy(pi)
            h_in.pop(ci).wait()

            buf = bufs[ci % NBUF]

            def row_body(rb, cs, buf=buf):
                new = list(cs)
                base = rb * U
                for u in range(U):
                    r = base + u
                    for j in range(NV):
                        acc = new[j] + buf[r, pl.ds(j * L, L)]
                        if store:
                            buf[r, pl.ds(j * L, L)] = acc
                        new[j] = acc
                return tuple(new)

            carries = lax.fori_loop(0, R // U, row_body, carries)
            if store:
                h_out[ci] = out_copy(ci)
        for ci in sorted(h_out):
            h_out[ci].wait()
        return carries

    zeros = tuple(jnp.zeros((L,), jnp.float32) for _ in range(NV))

    acc = stream(h * QTR, QTR // R, zeros, store=False)
    for j in range(NV):
        psum_v[pl.ds(j * L, L)] = acc[j]
    pltpu.sync_copy(psum_v, shared.at[s])
    plsc.subcore_barrier()
    pltpu.sync_copy(shared.at[lax.rem(s, NS // 2)], ppart_v)
    hvec = jnp.full((L,), h.astype(jnp.float32))
    carry = tuple(
        (ppart_v[pl.ds(j * L, L)] + acc[j]) * hvec for j in range(NV))

    stream(h * HALF, HALF // R, carry, store=True)


def kernel(x):
    return _cumsum_sc(x)


# speedup vs baseline: 1.0268x; 1.0268x over previous
import functools

import jax
import jax.numpy as jnp
from jax import lax
from jax.experimental import pallas as pl
from jax.experimental.pallas import tpu as pltpu
from jax.experimental.pallas import tpu_sc as plsc

ROWS = 8192
COLS = 2048
NC = 2
NS = 16
L = 16
CW = 128
NG = COLS // CW
NV = CW // L
HALF = ROWS // 2
QTR = ROWS // 4
R = 256
NBUF = 3
U = 8

_mesh = plsc.VectorSubcoreMesh(core_axis_name="c", subcore_axis_name="s")


@functools.partial(
    pl.kernel,
    out_type=jax.ShapeDtypeStruct((ROWS, COLS), jnp.float32),
    mesh=_mesh,
    scratch_types=[
        [pltpu.VMEM((R, CW), jnp.float32) for _ in range(NBUF)],
        pltpu.VMEM((CW,), jnp.float32),
        pltpu.VMEM((CW,), jnp.float32),
        pltpu.VMEM_SHARED((NS, CW), jnp.float32),
        pltpu.SemaphoreType.DMA,
        pltpu.SemaphoreType.DMA,
    ],
)
def _cumsum_sc(x_hbm, out_hbm, bufs, psum_v, ppart_v, shared, in_sem,
               out_sem):
    c = lax.axis_index("c")
    s = lax.axis_index("s")
    g = c * (NS // 2) + lax.rem(s, NS // 2)
    h = lax.div(s, NS // 2)
    c0 = g * CW

    def in_copy(row0, ci):
        return pltpu.async_copy(
            x_hbm.at[pl.ds(row0 + ci * R, R), pl.ds(c0, CW)],
            bufs[ci % NBUF], in_sem)

    def stream(row0, nchunk, carries, store, h_in=None):
        def out_copy(ci):
            return pltpu.async_copy(
                bufs[ci % NBUF],
                out_hbm.at[pl.ds(row0 + ci * R, R), pl.ds(c0, CW)], out_sem)

        prefetch = (NBUF - 2) if store else (NBUF - 1)
        if h_in is None:
            h_in = {}
            for ci in range(min(prefetch, nchunk)):
                h_in[ci] = in_copy(row0, ci)
        h_out = {}
        for ci in range(nchunk):
            pi = ci + prefetch
            if pi < nchunk and pi not in h_in:
                prev = pi - NBUF
                if store and prev >= 0:
                    h_out.pop(prev).wait()
                h_in[pi] = in_copy(row0, pi)
            h_in.pop(ci).wait()

            buf = bufs[ci % NBUF]

            def row_body(rb, cs, buf=buf):
                new = list(cs)
                base = rb * U
                for u in range(U):
                    r = base + u
                    for j in range(NV):
                        acc = new[j] + buf[r, pl.ds(j * L, L)]
                        if store:
                            buf[r, pl.ds(j * L, L)] = acc
                        new[j] = acc
                return tuple(new)

            carries = lax.fori_loop(0, R // U, row_body, carries)
            if store:
                h_out[ci] = out_copy(ci)
        for ci in sorted(h_out):
            h_out[ci].wait()
        return carries

    zeros = tuple(jnp.zeros((L,), jnp.float32) for _ in range(NV))

    acc = stream(h * QTR, QTR // R, zeros, store=False)
    for j in range(NV):
        psum_v[pl.ds(j * L, L)] = acc[j]
    pltpu.sync_copy(psum_v, shared.at[s])
    b_row0 = h * HALF
    b_pre = {0: in_copy(b_row0, 0)}
    plsc.subcore_barrier()
    pltpu.sync_copy(shared.at[lax.rem(s, NS // 2)], ppart_v)
    hvec = jnp.full((L,), h.astype(jnp.float32))
    carry = tuple(
        (ppart_v[pl.ds(j * L, L)] + acc[j]) * hvec for j in range(NV))

    stream(b_row0, HALF // R, carry, store=True, h_in=b_pre)


def kernel(x):
    return _cumsum_sc(x)
